# baseline (speedup 1.0000x reference)
import jax
import jax.numpy as jnp
from jax.experimental import pallas as pl
from jax.experimental.pallas import tpu as pltpu

VOCAB = 1_000_000
EMBED = 16
WINDOW = 2
BLK = 32_768


def _body(ctx_ref, emb0_ref, emb1_ref, w_ref, b_ref, out_ref):
    x = jnp.concatenate([emb0_ref[0, 0, :], emb1_ref[0, 0, :]])
    acc = jax.lax.dot_general(
        x[None, :], w_ref[...],
        (((1,), (1,)), ((), ())),
        preferred_element_type=jnp.float32,
    )
    out_ref[...] = acc + b_ref[...]


def kernel(context, emb_table, W, b):
    emb3 = emb_table.reshape(VOCAB, 1, EMBED)
    b2 = b.reshape(1, VOCAB)
    grid = (pl.cdiv(VOCAB, BLK),)
    grid_spec = pltpu.PrefetchScalarGridSpec(
        num_scalar_prefetch=1,
        grid=grid,
        in_specs=[
            pl.BlockSpec((1, 1, EMBED), lambda i, ctx: (ctx[0], 0, 0)),
            pl.BlockSpec((1, 1, EMBED), lambda i, ctx: (ctx[1], 0, 0)),
            pl.BlockSpec((BLK, EMBED * WINDOW), lambda i, ctx: (i, 0)),
            pl.BlockSpec((1, BLK), lambda i, ctx: (0, i)),
        ],
        out_specs=pl.BlockSpec((1, BLK), lambda i, ctx: (0, i)),
    )
    out = pl.pallas_call(
        _body,
        grid_spec=grid_spec,
        out_shape=jax.ShapeDtypeStruct((1, VOCAB), jnp.float32),
    )(context, emb3, emb3, W, b2)
    return out
